# baseline (speedup 1.0000x reference)
import functools

import jax
import jax.numpy as jnp
from jax import lax
from jax.experimental import pallas as pl
from jax.experimental.pallas import tpu as pltpu
from jax.experimental.pallas import tpu_sc as plsc

K_NN = 8
RADIUS = 0.5
BIG = 3.0e38
AGGW = 80


def _knn_block(pts_ref, kpt_ref, iw_ref, d0_ref):
    K = kpt_ref.shape[2]
    x = pts_ref[...]
    kpt = kpt_ref[0]
    x2 = jnp.sum(x * x, axis=1, keepdims=True)
    k2 = jnp.sum(kpt * kpt, axis=0, keepdims=True)
    xk = jnp.dot(x, kpt, preferred_element_type=jnp.float32)
    d2 = x2 - 2.0 * xk + k2

    iota_f = jax.lax.broadcasted_iota(jnp.int32, d2.shape, 1).astype(jnp.float32)
    d2m = d2
    idxs, ws = [], []
    for j in range(K_NN):
        mval = jnp.min(d2m, axis=1, keepdims=True)
        onehot = d2m == mval
        d2m = jnp.where(onehot, BIG, d2m)
        dj = jnp.maximum(mval, 0.0)
        if j == 0:
            d0_ref[...] = dj
        ws.append(1.0 / (jnp.sqrt(dj) + 1e-8))
        idxs.append(jnp.min(jnp.where(onehot, iota_f, BIG), axis=1, keepdims=True))
    wcat = jnp.concatenate(ws, axis=1)
    wcat = wcat / jnp.sum(wcat, axis=1, keepdims=True)
    iw_ref[...] = jnp.concatenate([wcat] + idxs, axis=1)


def _make_sc_agg(BP, P, K, NW):
    PW = BP // NW
    SUB = 128
    NCHUNK = AGGW // 16
    mesh = plsc.VectorSubcoreMesh(core_axis_name="c", subcore_axis_name="s")

    @functools.partial(
        pl.kernel, mesh=mesh,
        out_type=jax.ShapeDtypeStruct((BP, AGGW), jnp.float32),
        scratch_types=[
            pltpu.VMEM((K * AGGW,), jnp.float32),
            pltpu.VMEM((SUB, 16), jnp.float32),
            pltpu.VMEM((SUB, AGGW), jnp.float32),
        ],
    )
    def sc_agg(table_hbm, iw_hbm, agg_hbm, table_v, iw_v, out_v):
        info = plsc.get_sparse_core_info()
        nc = info.num_cores
        wid = lax.axis_index("s") * nc + lax.axis_index("c")
        pstart = wid * PW
        b = pstart // P
        pltpu.sync_copy(table_hbm.at[pl.ds(b * (K * AGGW), K * AGGW)], table_v)

        def chunk(cc, _):
            base = pstart + cc * SUB
            pltpu.sync_copy(iw_hbm.at[pl.ds(base, SUB)], iw_v)

            def point(p, _):
                iw = iw_v[p, :]
                accs = [jnp.zeros((16,), jnp.float32) for _ in range(NCHUNK)]
                for j in range(K_NN):
                    wj = iw[j]
                    row = iw[K_NN + j].astype(jnp.int32)
                    off = row * AGGW
                    for c in range(NCHUNK):
                        g = table_v[pl.ds(off + 16 * c, 16)]
                        accs[c] = accs[c] + wj * g
                for c in range(NCHUNK):
                    out_v[p, pl.ds(16 * c, 16)] = accs[c]
                return 0

            lax.fori_loop(0, SUB, point, 0)
            pltpu.sync_copy(out_v, agg_hbm.at[pl.ds(base, SUB)])
            return 0

        lax.fori_loop(0, PW // SUB, chunk, 0)

    return sc_agg


def _head_block(agg_ref, pts_ref, dirs_ref, d0_ref,
                w1a_ref, w1b_ref, b1_ref, ws_ref, bs_ref,
                wra_ref, wrb_ref, brgb_ref, out_ref):
    D = w1a_ref.shape[0]
    agg = agg_ref[...]
    agg_f = agg[:, :D]
    agg_p = agg[:, D:D + 3]
    x = pts_ref[...]
    rel = x - agg_p
    h = jnp.dot(agg_f, w1a_ref[...], preferred_element_type=jnp.float32)
    h = h + jnp.dot(rel, w1b_ref[...], preferred_element_type=jnp.float32)
    h = jnp.maximum(h + b1_ref[...], 0.0)

    z = jnp.dot(h, ws_ref[...], preferred_element_type=jnp.float32) + bs_ref[...] - 1.0
    sigma = jnp.maximum(z, 0.0) + jnp.log(1.0 + jnp.exp(-jnp.abs(z)))

    dn = dirs_ref[...]
    nrm = jnp.sqrt(jnp.sum(dn * dn, axis=1, keepdims=True))
    dirs = dn / (nrm + 1e-8)
    zr = (jnp.dot(h, wra_ref[...], preferred_element_type=jnp.float32)
          + jnp.dot(dirs, wrb_ref[...], preferred_element_type=jnp.float32)
          + brgb_ref[...])
    rgb = 1.0 / (1.0 + jnp.exp(-zr))

    maskf = jnp.where(d0_ref[...] < RADIUS * RADIUS, 1.0, 0.0)
    out_ref[...] = jnp.concatenate([sigma, rgb], axis=1) * maskf


def kernel(x, ray_dir, kp_pos, kp_feat, W1, b1, w_sigma, b_sigma, W_rgb, b_rgb, sample):
    B, T, R, S, _ = x.shape
    P = T * R * S
    BP = B * P
    K = kp_pos.shape[1]
    D = kp_feat.shape[2]
    H = W1.shape[1]

    pts = x.reshape(BP, 3)
    dirs = jnp.broadcast_to(ray_dir, (B, T, R, S, 3)).reshape(BP, 3)
    kpt = jnp.transpose(kp_pos, (0, 2, 1))

    BLK = 256
    nblk = P // BLK

    iw, d0 = pl.pallas_call(
        _knn_block,
        grid=(B, nblk),
        in_specs=[
            pl.BlockSpec((BLK, 3), lambda b, i: (b * nblk + i, 0)),
            pl.BlockSpec((1, 3, K), lambda b, i: (b, 0, 0)),
        ],
        out_specs=[
            pl.BlockSpec((BLK, 16), lambda b, i: (b * nblk + i, 0)),
            pl.BlockSpec((BLK, 1), lambda b, i: (b * nblk + i, 0)),
        ],
        out_shape=[
            jax.ShapeDtypeStruct((BP, 16), jnp.float32),
            jax.ShapeDtypeStruct((BP, 1), jnp.float32),
        ],
    )(pts, kpt)

    table = jnp.concatenate(
        [kp_feat, kp_pos,
         jnp.zeros((B, K, AGGW - D - 3), jnp.float32)], axis=2).reshape(B * K * AGGW)

    NW = 32
    agg = _make_sc_agg(BP, P, K, NW)(table, iw)

    W1a = W1[:D]
    W1b = W1[D:]
    Wra = W_rgb[:H]
    Wrb = W_rgb[H:]

    BLK2 = 512
    nblk2 = BP // BLK2
    full = lambda shape: pl.BlockSpec(shape, lambda i: (0,) * len(shape))
    out = pl.pallas_call(
        _head_block,
        grid=(nblk2,),
        in_specs=[
            pl.BlockSpec((BLK2, AGGW), lambda i: (i, 0)),
            pl.BlockSpec((BLK2, 3), lambda i: (i, 0)),
            pl.BlockSpec((BLK2, 3), lambda i: (i, 0)),
            pl.BlockSpec((BLK2, 1), lambda i: (i, 0)),
            full((D, H)), full((3, H)), full((1, H)),
            full((H, 1)), full((1, 1)),
            full((H, 3)), full((3, 3)), full((1, 3)),
        ],
        out_specs=pl.BlockSpec((BLK2, 4), lambda i: (i, 0)),
        out_shape=jax.ShapeDtypeStruct((BP, 4), jnp.float32),
    )(agg, pts, dirs, d0, W1a, W1b, b1.reshape(1, H), w_sigma,
      b_sigma.reshape(1, 1), Wra, Wrb, b_rgb.reshape(1, 3))

    return out.reshape(B, T, R, S, 4)


# speedup vs baseline: 1.1371x; 1.1371x over previous
import functools

import jax
import jax.numpy as jnp
from jax import lax
from jax.experimental import pallas as pl
from jax.experimental.pallas import tpu as pltpu
from jax.experimental.pallas import tpu_sc as plsc

K_NN = 8
RADIUS = 0.5
BIG = 3.0e38
AGGW = 80


def _knn_block(pts_ref, kpt_ref, iw_ref, d0_ref):
    x = pts_ref[...]
    kpt = kpt_ref[0]
    x2 = jnp.sum(x * x, axis=1, keepdims=True)
    k2 = jnp.sum(kpt * kpt, axis=0, keepdims=True)
    xk = jnp.dot(x, kpt, preferred_element_type=jnp.float32)
    d2 = x2 - 2.0 * xk + k2

    iota_f = jax.lax.broadcasted_iota(jnp.int32, d2.shape, 1).astype(jnp.float32)
    d2m = d2
    idxs, djs = [], []
    for j in range(K_NN):
        mval = jnp.min(d2m, axis=1, keepdims=True)
        onehot = d2m == mval
        d2m = jnp.where(onehot, BIG, d2m)
        djs.append(mval)
        idxs.append(jnp.min(jnp.where(onehot, iota_f, BIG), axis=1, keepdims=True))
    djcat = jnp.maximum(jnp.concatenate(djs, axis=1), 0.0)
    d0_ref[...] = djcat[:, :1]
    wcat = 1.0 / (jnp.sqrt(djcat) + 1e-8)
    wcat = wcat / jnp.sum(wcat, axis=1, keepdims=True)
    iw_ref[...] = jnp.concatenate([wcat] + idxs, axis=1)


def _make_sc_agg(CH, K, NW):
    PW = CH // NW
    SUB = 128
    NCHUNK = AGGW // 16
    mesh = plsc.VectorSubcoreMesh(core_axis_name="c", subcore_axis_name="s")

    @functools.partial(
        pl.kernel, mesh=mesh,
        out_type=jax.ShapeDtypeStruct((CH, AGGW), jnp.float32),
        scratch_types=[
            pltpu.VMEM((K * AGGW,), jnp.float32),
            pltpu.VMEM((SUB, 16), jnp.float32),
            pltpu.VMEM((SUB, AGGW), jnp.float32),
        ],
    )
    def sc_agg(table_hbm, iw_hbm, agg_hbm, table_v, iw_v, out_v):
        info = plsc.get_sparse_core_info()
        nc = info.num_cores
        wid = lax.axis_index("s") * nc + lax.axis_index("c")
        pstart = wid * PW
        pltpu.sync_copy(table_hbm, table_v)

        def chunk(cc, _):
            base = pstart + cc * SUB
            pltpu.sync_copy(iw_hbm.at[pl.ds(base, SUB)], iw_v)

            def point(p, _):
                iw = iw_v[p, :]
                accs = [jnp.zeros((16,), jnp.float32) for _ in range(NCHUNK)]
                for j in range(K_NN):
                    wj = iw[j]
                    row = iw[K_NN + j].astype(jnp.int32)
                    off = row * AGGW
                    for c in range(NCHUNK):
                        g = table_v[pl.ds(off + 16 * c, 16)]
                        accs[c] = accs[c] + wj * g
                for c in range(NCHUNK):
                    out_v[p, pl.ds(16 * c, 16)] = accs[c]
                return 0

            lax.fori_loop(0, SUB, point, 0)
            pltpu.sync_copy(out_v, agg_hbm.at[pl.ds(base, SUB)])
            return 0

        lax.fori_loop(0, PW // SUB, chunk, 0)

    return sc_agg


def _head_block(agg_ref, pts_ref, dirs_ref, d0_ref,
                w1a_ref, w1b_ref, b1_ref, ws_ref, bs_ref,
                wra_ref, wrb_ref, brgb_ref, out_ref):
    D = w1a_ref.shape[0]
    agg = agg_ref[...]
    agg_f = agg[:, :D]
    agg_p = agg[:, D:D + 3]
    x = pts_ref[...]
    rel = x - agg_p
    h = jnp.dot(agg_f, w1a_ref[...], preferred_element_type=jnp.float32)
    h = h + jnp.dot(rel, w1b_ref[...], preferred_element_type=jnp.float32)
    h = jnp.maximum(h + b1_ref[...], 0.0)

    z = jnp.dot(h, ws_ref[...], preferred_element_type=jnp.float32) + bs_ref[...] - 1.0
    sigma = jnp.maximum(z, 0.0) + jnp.log(1.0 + jnp.exp(-jnp.abs(z)))

    dn = dirs_ref[...]
    nrm = jnp.sqrt(jnp.sum(dn * dn, axis=1, keepdims=True))
    dirs = dn / (nrm + 1e-8)
    zr = (jnp.dot(h, wra_ref[...], preferred_element_type=jnp.float32)
          + jnp.dot(dirs, wrb_ref[...], preferred_element_type=jnp.float32)
          + brgb_ref[...])
    rgb = 1.0 / (1.0 + jnp.exp(-zr))

    maskf = jnp.where(d0_ref[...] < RADIUS * RADIUS, 1.0, 0.0)
    out_ref[...] = jnp.concatenate([sigma, rgb], axis=1) * maskf


def kernel(x, ray_dir, kp_pos, kp_feat, W1, b1, w_sigma, b_sigma, W_rgb, b_rgb, sample):
    B, T, R, S, _ = x.shape
    P = T * R * S
    BP = B * P
    K = kp_pos.shape[1]
    D = kp_feat.shape[2]
    H = W1.shape[1]

    pts = x.reshape(BP, 3)
    dirs = jnp.broadcast_to(ray_dir, (B, T, R, S, 3)).reshape(BP, 3)
    kpt = jnp.transpose(kp_pos, (0, 2, 1))

    table = jnp.concatenate(
        [kp_feat, kp_pos,
         jnp.zeros((B, K, AGGW - D - 3), jnp.float32)], axis=2).reshape(B, K * AGGW)

    W1a = W1[:D]
    W1b = W1[D:]
    Wra = W_rgb[:H]
    Wrb = W_rgb[H:]
    b1r = b1.reshape(1, H)
    bsr = b_sigma.reshape(1, 1)
    brr = b_rgb.reshape(1, 3)

    NW = 32
    BLK = 256
    CPB = 2 if P % (2 * NW * BLK) == 0 else 1
    CH = P // CPB
    nblk = CH // BLK
    sc_agg = _make_sc_agg(CH, K, NW)

    full = lambda shape: pl.BlockSpec(shape, lambda i: (0,) * len(shape))
    outs = []
    for b in range(B):
        for cc in range(CPB):
            lo = b * P + cc * CH
            pts_c = lax.slice(pts, (lo, 0), (lo + CH, 3))
            dirs_c = lax.slice(dirs, (lo, 0), (lo + CH, 3))

            iw, d0 = pl.pallas_call(
                _knn_block,
                grid=(nblk,),
                in_specs=[
                    pl.BlockSpec((BLK, 3), lambda i: (i, 0)),
                    pl.BlockSpec((1, 3, K), lambda i: (0, 0, 0)),
                ],
                out_specs=[
                    pl.BlockSpec((BLK, 16), lambda i: (i, 0)),
                    pl.BlockSpec((BLK, 1), lambda i: (i, 0)),
                ],
                out_shape=[
                    jax.ShapeDtypeStruct((CH, 16), jnp.float32),
                    jax.ShapeDtypeStruct((CH, 1), jnp.float32),
                ],
            )(pts_c, lax.slice(kpt, (b, 0, 0), (b + 1, 3, K)))

            agg = sc_agg(table[b], iw)

            out_c = pl.pallas_call(
                _head_block,
                grid=(CH // 512,),
                in_specs=[
                    pl.BlockSpec((512, AGGW), lambda i: (i, 0)),
                    pl.BlockSpec((512, 3), lambda i: (i, 0)),
                    pl.BlockSpec((512, 3), lambda i: (i, 0)),
                    pl.BlockSpec((512, 1), lambda i: (i, 0)),
                    full((D, H)), full((3, H)), full((1, H)),
                    full((H, 1)), full((1, 1)),
                    full((H, 3)), full((3, 3)), full((1, 3)),
                ],
                out_specs=pl.BlockSpec((512, 4), lambda i: (i, 0)),
                out_shape=jax.ShapeDtypeStruct((CH, 4), jnp.float32),
            )(agg, pts_c, dirs_c, d0, W1a, W1b, b1r, w_sigma, bsr, Wra, Wrb, brr)
            outs.append(out_c)

    out = jnp.concatenate(outs, axis=0)
    return out.reshape(B, T, R, S, 4)
